# Initial kernel scaffold; baseline (speedup 1.0000x reference)
import jax
import jax.numpy as jnp
from jax.experimental import pallas as pl


def kernel(pseudo_points, W):
    raise NotImplementedError("write your pallas kernel here")


# speedup vs baseline: 1.1818x; 1.1818x over previous
import functools

import jax
import jax.numpy as jnp
from jax import lax
from jax.experimental import pallas as pl
from jax.experimental.pallas import tpu as pltpu
from jax.experimental.pallas import tpu_sc as plsc

B = 4
N = 131072
GX, GY, GZ = 176, 200, 5
G = GX * GY * GZ
C_OUT = 16

NC, NS = 2, 16
TRASH = G
ACC_PER_TILE = 11008
G_PAD = NS * ACC_PER_TILE
ROWLEN = 64
CHUNK_ROWS = 32
PCHUNKS = 4
ZBLK = CHUNK_ROWS * ROWLEN
RD_ITER = 6


def _sc_body(pts_hbm, m_hbm, acc, pts_v, vals_v, zbuf, idx_v):
    c = lax.axis_index("c")
    s = lax.axis_index("s")
    lane = jnp.arange(16, dtype=jnp.int32)
    comp0 = jnp.zeros((16,), jnp.int32)
    comp1 = comp0 + 1
    comp2 = comp0 + 2
    zeros_f = jnp.zeros((16,), jnp.float32)
    zbase = s * ACC_PER_TILE

    def zb_body(i, carry):
        n = i * 16 + lane
        plsc.store_scatter(zbuf, [n >> 3, n & 7], zeros_f)
        return carry
    lax.fori_loop(0, ROWLEN * 8 // 16, zb_body, 0)

    def fill_idx(base):
        for g in range(4):
            plsc.store_scatter(idx_v, [lane + g * 16], base + g * 16 + lane)

    def pass_body(lb, carry):
        b = 2 * c + lb

        def vf_body(i, carry2):
            n = i * 16 + lane
            comp = n & 7
            plsc.store_scatter(
                vals_v, [n >> 9, (n >> 3) & 63, comp],
                jnp.where(comp == 3, jnp.float32(1.0), jnp.float32(0.0)))
            return carry2
        lax.fori_loop(0, CHUNK_ROWS * ROWLEN * 8 // 16, vf_body, 0)

        def zi_body(z, carry2):
            fill_idx(zbase + z * ROWLEN)
            pltpu.sync_copy(zbuf, acc.at[idx_v])
            return carry2
        lax.fori_loop(0, ACC_PER_TILE // ROWLEN, zi_body, 0)
        plsc.subcore_barrier()

        base_row = b * (N // ROWLEN) + s * (PCHUNKS * CHUNK_ROWS)

        def chunk_body(t, carry2):
            pltpu.sync_copy(
                pts_hbm.at[pl.ds(base_row + t * CHUNK_ROWS, CHUNK_ROWS)], pts_v)
            def pt_body(j, carry3):
                row = pts_v.at[j]
                vrow = vals_v.at[j]
                for k in range(ROWLEN // 16):
                    rows = lane + k * 16
                    gx = plsc.load_gather(row, [rows, comp0])
                    gy = plsc.load_gather(row, [rows, comp1])
                    gz = plsc.load_gather(row, [rows, comp2])
                    qx = gx / jnp.float32(0.4)
                    qy = (gy - jnp.float32(-40.0)) / jnp.float32(0.4)
                    qz = (gz - jnp.float32(-3.0)) / jnp.float32(0.8)
                    ix = qx.astype(jnp.int32)
                    iy = qy.astype(jnp.int32)
                    iz = qz.astype(jnp.int32)
                    valid = ((qx >= 0.0) & (qx < jnp.float32(GX))
                             & (qy >= 0.0) & (qy < jnp.float32(GY))
                             & (qz >= 0.0) & (qz < jnp.float32(GZ)))
                    seg = ix + GX * iy + (GX * GY) * iz
                    seg = jnp.where(valid, seg, TRASH)
                    plsc.store_scatter(idx_v, [rows], seg)
                    plsc.store_scatter(vrow, [rows, comp0], gx)
                    plsc.store_scatter(vrow, [rows, comp1], gy)
                    plsc.store_scatter(vrow, [rows, comp2], gz)
                pltpu.sync_copy(vrow, acc.at[idx_v], add=True)
                return carry3
            lax.fori_loop(0, CHUNK_ROWS, pt_body, 0)
            return carry2
        lax.fori_loop(0, PCHUNKS, chunk_body, 0)
        plsc.subcore_barrier()

        def rd_body(z, carry2):
            local = jnp.minimum(z * ZBLK, ACC_PER_TILE - ZBLK)
            def rj_body(j, carry3):
                fill_idx(zbase + local + j * ROWLEN)
                pltpu.sync_copy(acc.at[idx_v], vals_v.at[j])
                return carry3
            lax.fori_loop(0, CHUNK_ROWS, rj_body, 0)
            out_row = (b * G_PAD + zbase + local) // ROWLEN
            pltpu.sync_copy(vals_v, m_hbm.at[pl.ds(out_row, CHUNK_ROWS)])
            return carry2
        lax.fori_loop(0, RD_ITER, rd_body, 0)
        plsc.subcore_barrier()
        return carry
    lax.fori_loop(0, B // NC, pass_body, 0)


@functools.lru_cache(maxsize=1)
def _sc_voxelize():
    return pl.kernel(
        _sc_body,
        out_type=jax.ShapeDtypeStruct((B * G_PAD // ROWLEN, ROWLEN, 8),
                                      jnp.float32),
        mesh=plsc.VectorSubcoreMesh(
            core_axis_name="c", subcore_axis_name="s",
            num_cores=NC, num_subcores=NS),
        compiler_params=pltpu.CompilerParams(
            use_tc_tiling_on_sc=False, needs_layout_passes=False),
        scratch_types=[
            pltpu.VMEM_SHARED((G_PAD, 8), jnp.float32),
            pltpu.VMEM((CHUNK_ROWS, ROWLEN, 4), jnp.float32),
            pltpu.VMEM((CHUNK_ROWS, ROWLEN, 8), jnp.float32),
            pltpu.VMEM((ROWLEN, 8), jnp.float32),
            pltpu.VMEM((ROWLEN,), jnp.int32),
        ],
    )


def _tc_body(m_ref, s_ref, t_ref, o_ref):
    m = m_ref[...]
    num = jnp.maximum(
        jnp.dot(m, s_ref[...], preferred_element_type=jnp.float32), 0.0)
    cnt = jnp.dot(m, t_ref[...], preferred_element_type=jnp.float32)
    o_ref[...] = num / jnp.maximum(cnt, 1.0)


_TC_ROWS = B * G // 8
_TC_BLK = 1000


def _tc_finalize(mv, smat, tmat):
    return pl.pallas_call(
        _tc_body,
        grid=(_TC_ROWS // _TC_BLK,),
        in_specs=[
            pl.BlockSpec((_TC_BLK, 64), lambda i: (i, 0)),
            pl.BlockSpec((64, 128), lambda i: (0, 0)),
            pl.BlockSpec((64, 128), lambda i: (0, 0)),
        ],
        out_specs=pl.BlockSpec((_TC_BLK, 128), lambda i: (i, 0)),
        out_shape=jax.ShapeDtypeStruct((_TC_ROWS, 128), jnp.float32),
    )(mv, smat, tmat)


def kernel(pseudo_points, W):
    pts = pseudo_points.reshape(B * N // ROWLEN, ROWLEN, 4)
    m_pad = _sc_voxelize()(pts).reshape(B * G_PAD, 8)
    m = jnp.concatenate([m_pad[b * G_PAD: b * G_PAD + G] for b in range(B)],
                        axis=0)
    eye8 = jnp.eye(8, dtype=jnp.float32)
    w8 = jnp.concatenate([W, jnp.zeros((5, C_OUT), jnp.float32)], axis=0)
    smat = jnp.kron(eye8, w8)
    t8 = jnp.concatenate(
        [jnp.zeros((3, C_OUT), jnp.float32), jnp.ones((1, C_OUT), jnp.float32),
         jnp.zeros((4, C_OUT), jnp.float32)], axis=0)
    tmat = jnp.kron(eye8, t8)
    out = _tc_finalize(m.reshape(_TC_ROWS, 64), smat, tmat)
    return out.reshape(B * G, C_OUT)
